# baseline (speedup 1.0000x reference)
import functools

import jax
import jax.numpy as jnp
from jax import lax
from jax.experimental import pallas as pl
from jax.experimental.pallas import tpu as pltpu
import jax.experimental.pallas.tpu_sc as plsc

N = 10000
E = 320000
D = 128
NC = 2
NS = 16
NW = NC * NS
EB = 128
NBUF = 2
IB = 4

ITC0 = 40
ITC1 = 40
HALVES = 2
TOT_CH = NS * (ITC0 + ITC1)
E_PAD = HALVES * TOT_CH * EB
ITERS = HALVES * TOT_CH // NW
N_PAD = 10112
RPT = N_PAD // NS

BM = 2000
GRID = N // BM


def _sc_mesh():
    return plsc.VectorSubcoreMesh(core_axis_name="c", subcore_axis_name="s",
                                  num_cores=NC, num_subcores=NS)


@functools.lru_cache(maxsize=None)
def _make_deg_kernel():
    @functools.partial(
        pl.kernel,
        out_type=jax.ShapeDtypeStruct((NC, N_PAD, D), jnp.float32),
        mesh=_sc_mesh(),
        scratch_types=[
            pltpu.VMEM((ITERS, EB), jnp.int32),
            pltpu.VMEM((EB, D), jnp.float32),
            pltpu.VMEM_SHARED((N_PAD, D), jnp.float32),
            pltpu.SemaphoreType.DMA((IB,)),
        ],
    )
    def _deg_kernel(row_hbm, ones_hbm, zeros_hbm, out_hbm,
                    rowm, onesv, acc, ssem):
        c = lax.axis_index("c")
        s = lax.axis_index("s")
        wid = s * NC + c
        pltpu.sync_copy(row_hbm.at[pl.ds(wid * ITERS, ITERS)], rowm)
        pltpu.sync_copy(zeros_hbm, acc.at[pl.ds(s * RPT, RPT)])
        pltpu.sync_copy(ones_hbm, onesv)
        plsc.subcore_barrier()

        def body(sup, carry):
            for b in range(IB):
                k = sup * IB + b
                pltpu.async_copy(onesv, acc.at[rowm.at[k]], ssem.at[b],
                                 add=True)
            for b in range(IB):
                k = sup * IB + b
                pltpu.make_async_copy(onesv, acc.at[rowm.at[k]],
                                      ssem.at[b]).wait()
            return carry

        lax.fori_loop(0, ITERS // IB, body, 0)
        plsc.subcore_barrier()
        pltpu.sync_copy(acc.at[pl.ds(s * RPT, RPT)],
                        out_hbm.at[c, pl.ds(s * RPT, RPT)])

    return _deg_kernel


@functools.lru_cache(maxsize=None)
def _make_agg_kernel():
    @functools.partial(
        pl.kernel,
        out_type=jax.ShapeDtypeStruct((NC, N_PAD, D), jnp.float32),
        mesh=_sc_mesh(),
        scratch_types=(
            [pltpu.VMEM((EB,), jnp.int32) for _ in range(2 * IB)]
            + [pltpu.VMEM((EB, D), jnp.float32) for _ in range(NBUF)]
            + [
                pltpu.VMEM_SHARED((N_PAD, D), jnp.float32),
                pltpu.SemaphoreType.DMA((IB,)),
                pltpu.SemaphoreType.DMA((NBUF,)),
                pltpu.SemaphoreType.DMA((NBUF,)),
            ]
        ),
    )
    def _agg_kernel(hs_hbm, row_hbm, col_hbm, zeros_hbm, out_hbm, *rest):
        rowb = rest[:IB]
        colb = rest[IB:2 * IB]
        gbufs = rest[2 * IB:2 * IB + NBUF]
        acc, isem, gsem, ssem = rest[2 * IB + NBUF:2 * IB + NBUF + 4]
        c = lax.axis_index("c")
        s = lax.axis_index("s")
        base_ch = jnp.where(c == 0, s * ITC0, NS * ITC0 + s * ITC1)
        tg = jnp.where(c == 0, ITC0 // 4, ITC1 // 4)

        def idx_start(k, j):
            pltpu.async_copy(row_hbm.at[base_ch + k], rowb[j], isem.at[j])
            pltpu.async_copy(col_hbm.at[base_ch + k], colb[j], isem.at[j])

        def idx_wait(k, j):
            pltpu.make_async_copy(row_hbm.at[base_ch + k], rowb[j],
                                  isem.at[j]).wait()
            pltpu.make_async_copy(col_hbm.at[base_ch + k], colb[j],
                                  isem.at[j]).wait()

        def gather_start(b, j):
            pltpu.async_copy(hs_hbm.at[colb[j]], gbufs[b], gsem.at[b])

        def gather_wait(b, j):
            pltpu.make_async_copy(hs_hbm.at[colb[j]], gbufs[b],
                                  gsem.at[b]).wait()

        def scat_start(b, j):
            pltpu.async_copy(gbufs[b], acc.at[rowb[j]], ssem.at[b], add=True)

        def scat_wait(b, j):
            pltpu.make_async_copy(gbufs[b], acc.at[rowb[j]],
                                  ssem.at[b]).wait()

        pltpu.sync_copy(zeros_hbm, acc.at[pl.ds(s * RPT, RPT)])
        plsc.subcore_barrier()

        for j in range(IB):
            idx_start(j, j)
        for b in range(NBUF):
            idx_wait(b, b)
            gather_start(b, b)

        def body(t, carry):
            k0 = 4 * t

            def half(base, j0, j1, jn0, jn1, refill_ok, prefetch_ok):
                gather_wait(0, j0)
                scat_start(0, j0)
                gather_wait(1, j1)
                scat_start(1, j1)

                def refill():
                    scat_wait(0, j0)
                    idx_wait(base + 2, jn0)
                    gather_start(0, jn0)
                    scat_wait(1, j1)
                    idx_wait(base + 3, jn1)
                    gather_start(1, jn1)

                if refill_ok is None:
                    refill()
                else:
                    pl.when(refill_ok)(refill)

                @pl.when(prefetch_ok)
                def _():
                    idx_start(base + 4, j0)
                    idx_start(base + 5, j1)

            not_last = t < tg - 1
            half(k0, 0, 1, 2, 3, None, not_last)
            half(k0 + 2, 2, 3, 0, 1, not_last, not_last)
            return carry

        lax.fori_loop(0, tg, body, 0)
        scat_wait(0, 2)
        scat_wait(1, 3)
        plsc.subcore_barrier()
        pltpu.sync_copy(acc.at[pl.ds(s * RPT, RPT)],
                        out_hbm.at[c, pl.ds(s * RPT, RPT)])

    return _agg_kernel


def _prep_body(d0_ref, d1_ref, x_ref, dis_ref, h0s_ref):
    deg = d0_ref[:, 0:1] + d1_ref[:, 0:1]
    dis = jnp.where(deg > 0, lax.rsqrt(jnp.maximum(deg, 1e-12)), 0.0)
    dis_b = jnp.broadcast_to(dis, (BM, D))
    dis_ref[...] = dis_b
    h0s_ref[...] = x_ref[...] * dis_b


def _prep_call(d0, d1, x):
    return pl.pallas_call(
        _prep_body,
        grid=(GRID,),
        in_specs=[
            pl.BlockSpec((BM, D), lambda i: (i, 0)),
            pl.BlockSpec((BM, D), lambda i: (i, 0)),
            pl.BlockSpec((BM, D), lambda i: (i, 0)),
        ],
        out_specs=[
            pl.BlockSpec((BM, D), lambda i: (i, 0)),
            pl.BlockSpec((BM, D), lambda i: (i, 0)),
        ],
        out_shape=[
            jax.ShapeDtypeStruct((N, D), jnp.float32),
            jax.ShapeDtypeStruct((N, D), jnp.float32),
        ],
    )(d0, d1, x)


def _leaky(v):
    return jnp.maximum(v, 0.2 * v)


def _dense_body(p0_ref, p1_ref, p2_ref, p3_ref, h_ref, dis_ref, wg_ref,
                bg_ref, wi_ref, bi_ref, hn_ref, hns_ref):
    dis = dis_ref[...]
    ha = ((p0_ref[...] + p1_ref[...])
          + (p2_ref[...] + p3_ref[...])) * dis
    h = h_ref[...]
    a = _leaky(jnp.dot(ha, wg_ref[...],
                       preferred_element_type=jnp.float32) + bg_ref[...])
    b = _leaky(jnp.dot(h * ha, wi_ref[...],
                       preferred_element_type=jnp.float32) + bi_ref[...])
    u = a + b
    sq = jnp.sum(u * u, axis=-1, keepdims=True)
    hn = u * lax.rsqrt(jnp.maximum(sq, 1e-12))
    hn_ref[...] = hn
    hns_ref[...] = hn * dis


def _dense_call(p0, p1, p2, p3, h, dis, wg, bg, wi, bi):
    full = lambda i: (0, 0)
    blk = lambda i: (i, 0)
    return pl.pallas_call(
        _dense_body,
        grid=(GRID,),
        in_specs=[
            pl.BlockSpec((BM, D), blk),
            pl.BlockSpec((BM, D), blk),
            pl.BlockSpec((BM, D), blk),
            pl.BlockSpec((BM, D), blk),
            pl.BlockSpec((BM, D), blk),
            pl.BlockSpec((BM, D), blk),
            pl.BlockSpec((D, D), full),
            pl.BlockSpec((1, D), full),
            pl.BlockSpec((D, D), full),
            pl.BlockSpec((1, D), full),
        ],
        out_specs=[
            pl.BlockSpec((BM, D), blk),
            pl.BlockSpec((BM, D), blk),
        ],
        out_shape=[
            jax.ShapeDtypeStruct((N, D), jnp.float32),
            jax.ShapeDtypeStruct((N, D), jnp.float32),
        ],
    )(p0, p1, p2, p3, h, dis, wg, bg, wi, bi)


@jax.jit
def _run(x, edge_index, Wg0, bg0, Wi0, bi0, Wg1, bg1, Wi1, bi1,
         Wg2, bg2, Wi2, bi2):
    row = edge_index[0]
    col = edge_index[1]
    pad = max(E_PAD - E, 0)
    row_p = jnp.concatenate([row, jnp.full((max(pad, 0),), N, jnp.int32)])[:E_PAD]
    col_p = jnp.concatenate([col, jnp.zeros((max(pad, 0),), jnp.int32)])[:E_PAD]
    row3 = row_p.reshape(HALVES * TOT_CH, EB)
    col3 = col_p.reshape(HALVES * TOT_CH, EB)
    rowh = [row3[i * TOT_CH:(i + 1) * TOT_CH] for i in range(HALVES)]
    colh = [col3[i * TOT_CH:(i + 1) * TOT_CH] for i in range(HALVES)]

    zerosD = jnp.zeros((RPT, D), jnp.float32)
    onesD = jnp.ones((EB, D), jnp.float32)

    deg_parts = _make_deg_kernel()(row3, onesD, zerosD)
    dis, hs = _prep_call(deg_parts[0, :N], deg_parts[1, :N], x)

    params = [(Wg0, bg0, Wi0, bi0), (Wg1, bg1, Wi1, bi1), (Wg2, bg2, Wi2, bi2)]
    h = x
    outs = [x]
    for (Wg, bg, Wi, bi) in params:
        pa = _make_agg_kernel()(hs, rowh[0], colh[0], zerosD)
        pb = _make_agg_kernel()(hs, rowh[1], colh[1], zerosD)
        h, hs = _dense_call(pa[0, :N], pa[1, :N], pb[0, :N], pb[1, :N],
                            h, dis, Wg, bg.reshape(1, D), Wi,
                            bi.reshape(1, D))
        outs.append(h)
    return jnp.concatenate(outs, axis=-1)


def kernel(x, edge_index, Wg0, bg0, Wi0, bi0, Wg1, bg1, Wi1, bi1,
           Wg2, bg2, Wi2, bi2):
    return _run(x, edge_index, Wg0, bg0, Wi0, bi0, Wg1, bg1, Wi1, bi1,
                Wg2, bg2, Wi2, bi2)


# speedup vs baseline: 2.6675x; 2.6675x over previous
import functools

import jax
import jax.numpy as jnp
from jax import lax
from jax.experimental import pallas as pl
from jax.experimental.pallas import tpu as pltpu
import jax.experimental.pallas.tpu_sc as plsc

N = 10000
E = 320000
D = 128
NC = 2
NS = 16
NW = NC * NS
EB = 128
NBUF = 2
IB = 4

ITC0 = 80
ITC1 = 80
TOT_CH = NS * (ITC0 + ITC1)
E_PAD = TOT_CH * EB
ITERS = TOT_CH // NW
N_PAD = 10112
RPT = N_PAD // NS
NZ = 8
NTRASH = N_PAD - N - NZ

BM = 2000
GRID = N // BM


def _sc_mesh():
    return plsc.VectorSubcoreMesh(core_axis_name="c", subcore_axis_name="s",
                                  num_cores=NC, num_subcores=NS)


@functools.lru_cache(maxsize=None)
def _make_deg_kernel():
    @functools.partial(
        pl.kernel,
        out_type=jax.ShapeDtypeStruct((NC, N_PAD, D), jnp.float32),
        mesh=_sc_mesh(),
        scratch_types=[
            pltpu.VMEM((ITERS, EB), jnp.int32),
            pltpu.VMEM((EB, D), jnp.float32),
            pltpu.VMEM_SHARED((N_PAD, D), jnp.float32),
            pltpu.SemaphoreType.DMA((IB,)),
        ],
    )
    def _deg_kernel(row_hbm, ones_hbm, zeros_hbm, out_hbm,
                    rowm, onesv, acc, ssem):
        c = lax.axis_index("c")
        s = lax.axis_index("s")
        wid = s * NC + c
        pltpu.sync_copy(row_hbm.at[pl.ds(wid * ITERS, ITERS)], rowm)
        pltpu.sync_copy(zeros_hbm, acc.at[pl.ds(s * RPT, RPT)])
        pltpu.sync_copy(ones_hbm, onesv)
        plsc.subcore_barrier()

        def body(sup, carry):
            for b in range(IB):
                k = sup * IB + b
                pltpu.async_copy(onesv, acc.at[rowm.at[k]], ssem.at[b],
                                 add=True)
            for b in range(IB):
                k = sup * IB + b
                pltpu.make_async_copy(onesv, acc.at[rowm.at[k]],
                                      ssem.at[b]).wait()
            return carry

        lax.fori_loop(0, ITERS // IB, body, 0)
        plsc.subcore_barrier()
        pltpu.sync_copy(acc.at[pl.ds(s * RPT, RPT)],
                        out_hbm.at[c, pl.ds(s * RPT, RPT)])

    return _deg_kernel


@functools.lru_cache(maxsize=None)
def _make_agg_kernel():
    @functools.partial(
        pl.kernel,
        out_type=jax.ShapeDtypeStruct((NC, N_PAD, D), jnp.float32),
        mesh=_sc_mesh(),
        scratch_types=(
            [pltpu.VMEM((EB,), jnp.int32) for _ in range(2 * IB)]
            + [pltpu.VMEM((EB, D), jnp.float32) for _ in range(NBUF)]
            + [
                pltpu.VMEM_SHARED((N_PAD, D), jnp.float32),
                pltpu.SemaphoreType.DMA((IB,)),
                pltpu.SemaphoreType.DMA((NBUF,)),
                pltpu.SemaphoreType.DMA((NBUF,)),
            ]
        ),
    )
    def _agg_kernel(hs_hbm, row_hbm, col_hbm, zeros_hbm, out_hbm, *rest):
        rowb = rest[:IB]
        colb = rest[IB:2 * IB]
        gbufs = rest[2 * IB:2 * IB + NBUF]
        acc, isem, gsem, ssem = rest[2 * IB + NBUF:2 * IB + NBUF + 4]
        c = lax.axis_index("c")
        s = lax.axis_index("s")
        base_ch = jnp.where(c == 0, s * ITC0, NS * ITC0 + s * ITC1)
        tg = jnp.where(c == 0, ITC0 // 4, ITC1 // 4)

        def idx_start(k, j):
            pltpu.async_copy(row_hbm.at[base_ch + k], rowb[j], isem.at[j])
            pltpu.async_copy(col_hbm.at[base_ch + k], colb[j], isem.at[j])

        def idx_wait(k, j):
            pltpu.make_async_copy(row_hbm.at[base_ch + k], rowb[j],
                                  isem.at[j]).wait()
            pltpu.make_async_copy(col_hbm.at[base_ch + k], colb[j],
                                  isem.at[j]).wait()

        def gather_start(b, j):
            pltpu.async_copy(hs_hbm.at[colb[j]], gbufs[b], gsem.at[b])

        def gather_wait(b, j):
            pltpu.make_async_copy(hs_hbm.at[colb[j]], gbufs[b],
                                  gsem.at[b]).wait()

        def scat_start(b, j):
            pltpu.async_copy(gbufs[b], acc.at[rowb[j]], ssem.at[b], add=True)

        def scat_wait(b, j):
            pltpu.make_async_copy(gbufs[b], acc.at[rowb[j]],
                                  ssem.at[b]).wait()

        pltpu.sync_copy(zeros_hbm, acc.at[pl.ds(s * RPT, RPT)])
        plsc.subcore_barrier()

        for j in range(IB):
            idx_start(j, j)
        for b in range(NBUF):
            idx_wait(b, b)
            gather_start(b, b)

        def body(t, carry):
            k0 = 4 * t

            def half(base, j0, j1, jn0, jn1, refill_ok, prefetch_ok):
                gather_wait(0, j0)
                scat_start(0, j0)
                gather_wait(1, j1)
                scat_start(1, j1)

                def refill():
                    scat_wait(0, j0)
                    idx_wait(base + 2, jn0)
                    gather_start(0, jn0)
                    scat_wait(1, j1)
                    idx_wait(base + 3, jn1)
                    gather_start(1, jn1)

                if refill_ok is None:
                    refill()
                else:
                    pl.when(refill_ok)(refill)

                @pl.when(prefetch_ok)
                def _():
                    idx_start(base + 4, j0)
                    idx_start(base + 5, j1)

            not_last = t < tg - 1
            half(k0, 0, 1, 2, 3, None, not_last)
            half(k0 + 2, 2, 3, 0, 1, not_last, not_last)
            return carry

        lax.fori_loop(0, tg, body, 0)
        scat_wait(0, 2)
        scat_wait(1, 3)
        plsc.subcore_barrier()
        pltpu.sync_copy(acc.at[pl.ds(s * RPT, RPT)],
                        out_hbm.at[c, pl.ds(s * RPT, RPT)])

    return _agg_kernel


def _prep_body(d0_ref, d1_ref, x_ref, dis_ref, h0s_ref):
    deg = d0_ref[:, 0:1] + d1_ref[:, 0:1]
    dis = jnp.where(deg > 0, lax.rsqrt(jnp.maximum(deg, 1e-12)), 0.0)
    dis_b = jnp.broadcast_to(dis, (BM, D))
    dis_ref[...] = dis_b
    h0s_ref[...] = x_ref[...] * dis_b


def _prep_call(d0, d1, x):
    return pl.pallas_call(
        _prep_body,
        grid=(GRID,),
        in_specs=[
            pl.BlockSpec((BM, D), lambda i: (i, 0)),
            pl.BlockSpec((BM, D), lambda i: (i, 0)),
            pl.BlockSpec((BM, D), lambda i: (i, 0)),
        ],
        out_specs=[
            pl.BlockSpec((BM, D), lambda i: (i, 0)),
            pl.BlockSpec((BM, D), lambda i: (i, 0)),
        ],
        out_shape=[
            jax.ShapeDtypeStruct((N, D), jnp.float32),
            jax.ShapeDtypeStruct((N, D), jnp.float32),
        ],
    )(d0, d1, x)


def _leaky(v):
    return jnp.maximum(v, 0.2 * v)


def _dense_body(p0_ref, p1_ref, h_ref, dis_ref, wg_ref,
                bg_ref, wi_ref, bi_ref, hn_ref, hns_ref):
    dis = dis_ref[...]
    ha = (p0_ref[...] + p1_ref[...]) * dis
    h = h_ref[...]
    a = _leaky(jnp.dot(ha, wg_ref[...],
                       preferred_element_type=jnp.float32) + bg_ref[...])
    b = _leaky(jnp.dot(h * ha, wi_ref[...],
                       preferred_element_type=jnp.float32) + bi_ref[...])
    u = a + b
    sq = jnp.sum(u * u, axis=-1, keepdims=True)
    hn = u * lax.rsqrt(jnp.maximum(sq, 1e-12))
    hn_ref[...] = hn
    hns_ref[...] = hn * dis


def _dense_call(p0, p1, h, dis, wg, bg, wi, bi):
    full = lambda i: (0, 0)
    blk = lambda i: (i, 0)
    return pl.pallas_call(
        _dense_body,
        grid=(GRID,),
        in_specs=[
            pl.BlockSpec((BM, D), blk),
            pl.BlockSpec((BM, D), blk),
            pl.BlockSpec((BM, D), blk),
            pl.BlockSpec((BM, D), blk),
            pl.BlockSpec((D, D), full),
            pl.BlockSpec((1, D), full),
            pl.BlockSpec((D, D), full),
            pl.BlockSpec((1, D), full),
        ],
        out_specs=[
            pl.BlockSpec((BM, D), blk),
            pl.BlockSpec((BM, D), blk),
        ],
        out_shape=[
            jax.ShapeDtypeStruct((N, D), jnp.float32),
            jax.ShapeDtypeStruct((N, D), jnp.float32),
        ],
    )(p0, p1, h, dis, wg, bg, wi, bi)


@jax.jit
def _run(x, edge_index, Wg0, bg0, Wi0, bi0, Wg1, bg1, Wi1, bi1,
         Wg2, bg2, Wi2, bi2):
    row = edge_index[0]
    col = edge_index[1]
    pad = E_PAD - E
    ar = jnp.arange(pad, dtype=jnp.int32)
    row_p = jnp.concatenate([row, N + NZ + ar % NTRASH])
    col_p = jnp.concatenate([col, N + ar % NZ])
    row3 = row_p.reshape(TOT_CH, EB)
    col3 = col_p.reshape(TOT_CH, EB)

    zerosD = jnp.zeros((RPT, D), jnp.float32)
    onesD = jnp.ones((EB, D), jnp.float32)

    deg_parts = _make_deg_kernel()(row3, onesD, zerosD)
    dis, hs = _prep_call(deg_parts[0, :N], deg_parts[1, :N], x)

    params = [(Wg0, bg0, Wi0, bi0), (Wg1, bg1, Wi1, bi1), (Wg2, bg2, Wi2, bi2)]
    h = x
    outs = [x]
    for (Wg, bg, Wi, bi) in params:
        hs_ext = jnp.concatenate([hs, jnp.zeros((NZ, D), jnp.float32)])
        parts = _make_agg_kernel()(hs_ext, row3, col3, zerosD)
        h, hs = _dense_call(parts[0, :N], parts[1, :N], h, dis,
                            Wg, bg.reshape(1, D), Wi, bi.reshape(1, D))
        outs.append(h)
    return jnp.concatenate(outs, axis=-1)


def kernel(x, edge_index, Wg0, bg0, Wi0, bi0, Wg1, bg1, Wi1, bi1,
           Wg2, bg2, Wi2, bi2):
    return _run(x, edge_index, Wg0, bg0, Wi0, bi0, Wg1, bg1, Wi1, bi1,
                Wg2, bg2, Wi2, bi2)


# speedup vs baseline: 2.9390x; 1.1018x over previous
import functools

import jax
import jax.numpy as jnp
from jax import lax
from jax.experimental import pallas as pl
from jax.experimental.pallas import tpu as pltpu
import jax.experimental.pallas.tpu_sc as plsc

N = 10000
E = 320000
D = 128
NC = 2
NS = 16
NW = NC * NS
EB = 128
NBUF = 2
IB = 4

ITC0 = 80
ITC1 = 80
TOT_CH = NS * (ITC0 + ITC1)
E_PAD = TOT_CH * EB
ITERS = TOT_CH // NW
N_PAD = 10112
RPT = N_PAD // NS
NZ = 8
NTRASH = N_PAD - N - NZ

BM = 2000
GRID = N // BM


def _sc_mesh():
    return plsc.VectorSubcoreMesh(core_axis_name="c", subcore_axis_name="s",
                                  num_cores=NC, num_subcores=NS)


@functools.lru_cache(maxsize=None)
def _make_deg_kernel():
    @functools.partial(
        pl.kernel,
        out_type=jax.ShapeDtypeStruct((NW, N_PAD), jnp.float32),
        mesh=_sc_mesh(),
        compiler_params=pltpu.CompilerParams(needs_layout_passes=False),
        scratch_types=[
            pltpu.VMEM((ITERS, EB), jnp.int32),
            pltpu.VMEM((N_PAD,), jnp.float32),
        ],
    )
    def _deg_kernel(row_hbm, out_hbm, rowm, hist):
        c = lax.axis_index("c")
        s = lax.axis_index("s")
        wid = s * NC + c
        pltpu.sync_copy(row_hbm.at[pl.ds(wid * ITERS, ITERS)], rowm)

        def zbody(j, carry):
            hist[pl.ds(16 * j, 16)] = jnp.zeros((16,), jnp.float32)
            return carry

        lax.fori_loop(0, N_PAD // 16, zbody, 0)
        ones = jnp.ones((16,), jnp.float32)

        def body(k, carry):
            for j in range(EB // 16):
                i16 = rowm[k, pl.ds(16 * j, 16)]
                plsc.addupdate_scatter(hist, [i16], ones)
            return carry

        lax.fori_loop(0, ITERS, body, 0)
        pltpu.sync_copy(hist, out_hbm.at[wid])

    return _deg_kernel


@functools.lru_cache(maxsize=None)
def _make_agg_kernel():
    @functools.partial(
        pl.kernel,
        out_type=jax.ShapeDtypeStruct((NC, N_PAD, D), jnp.float32),
        mesh=_sc_mesh(),
        scratch_types=(
            [pltpu.VMEM((EB,), jnp.int32) for _ in range(2 * IB)]
            + [pltpu.VMEM((EB, D), jnp.float32) for _ in range(NBUF)]
            + [
                pltpu.VMEM_SHARED((N_PAD, D), jnp.float32),
                pltpu.SemaphoreType.DMA((IB,)),
                pltpu.SemaphoreType.DMA((NBUF,)),
                pltpu.SemaphoreType.DMA((NBUF,)),
            ]
        ),
    )
    def _agg_kernel(hs_hbm, row_hbm, col_hbm, zeros_hbm, out_hbm, *rest):
        rowb = rest[:IB]
        colb = rest[IB:2 * IB]
        gbufs = rest[2 * IB:2 * IB + NBUF]
        acc, isem, gsem, ssem = rest[2 * IB + NBUF:2 * IB + NBUF + 4]
        c = lax.axis_index("c")
        s = lax.axis_index("s")
        base_ch = jnp.where(c == 0, s * ITC0, NS * ITC0 + s * ITC1)
        tg = jnp.where(c == 0, ITC0 // 4, ITC1 // 4)

        def idx_start(k, j):
            pltpu.async_copy(row_hbm.at[base_ch + k], rowb[j], isem.at[j])
            pltpu.async_copy(col_hbm.at[base_ch + k], colb[j], isem.at[j])

        def idx_wait(k, j):
            pltpu.make_async_copy(row_hbm.at[base_ch + k], rowb[j],
                                  isem.at[j]).wait()
            pltpu.make_async_copy(col_hbm.at[base_ch + k], colb[j],
                                  isem.at[j]).wait()

        def gather_start(b, j):
            pltpu.async_copy(hs_hbm.at[colb[j]], gbufs[b], gsem.at[b])

        def gather_wait(b, j):
            pltpu.make_async_copy(hs_hbm.at[colb[j]], gbufs[b],
                                  gsem.at[b]).wait()

        def scat_start(b, j):
            pltpu.async_copy(gbufs[b], acc.at[rowb[j]], ssem.at[b], add=True)

        def scat_wait(b, j):
            pltpu.make_async_copy(gbufs[b], acc.at[rowb[j]],
                                  ssem.at[b]).wait()

        pltpu.sync_copy(zeros_hbm, acc.at[pl.ds(s * RPT, RPT)])
        plsc.subcore_barrier()

        for j in range(IB):
            idx_start(j, j)
        for b in range(NBUF):
            idx_wait(b, b)
            gather_start(b, b)

        def body(t, carry):
            k0 = 4 * t

            def half(base, j0, j1, jn0, jn1, refill_ok, prefetch_ok):
                gather_wait(0, j0)
                scat_start(0, j0)
                gather_wait(1, j1)
                scat_start(1, j1)

                def refill():
                    scat_wait(0, j0)
                    idx_wait(base + 2, jn0)
                    gather_start(0, jn0)
                    scat_wait(1, j1)
                    idx_wait(base + 3, jn1)
                    gather_start(1, jn1)

                if refill_ok is None:
                    refill()
                else:
                    pl.when(refill_ok)(refill)

                @pl.when(prefetch_ok)
                def _():
                    idx_start(base + 4, j0)
                    idx_start(base + 5, j1)

            not_last = t < tg - 1
            half(k0, 0, 1, 2, 3, None, not_last)
            half(k0 + 2, 2, 3, 0, 1, not_last, not_last)
            return carry

        lax.fori_loop(0, tg, body, 0)
        scat_wait(0, 2)
        scat_wait(1, 3)
        plsc.subcore_barrier()
        pltpu.sync_copy(acc.at[pl.ds(s * RPT, RPT)],
                        out_hbm.at[c, pl.ds(s * RPT, RPT)])

    return _agg_kernel


def _prep_body(d_ref, x_ref, dis_ref, h0s_ref):
    deg = lax.dot_general(d_ref[...], jnp.ones((NW, 1), jnp.float32),
                          (((0,), (0,)), ((), ())),
                          preferred_element_type=jnp.float32)[:N]
    dis = jnp.where(deg > 0, lax.rsqrt(jnp.maximum(deg, 1e-12)), 0.0)
    dis_b = jnp.broadcast_to(dis, (N, D))
    dis_ref[...] = dis_b
    h0s_ref[...] = x_ref[...] * dis_b


def _prep_call(d, x):
    return pl.pallas_call(
        _prep_body,
        out_shape=[
            jax.ShapeDtypeStruct((N, D), jnp.float32),
            jax.ShapeDtypeStruct((N, D), jnp.float32),
        ],
    )(d, x)


def _leaky(v):
    return jnp.maximum(v, 0.2 * v)


def _dense_body(p0_ref, p1_ref, h_ref, dis_ref, wg_ref,
                bg_ref, wi_ref, bi_ref, hn_ref, hns_ref):
    dis = dis_ref[...]
    ha = (p0_ref[...] + p1_ref[...]) * dis
    h = h_ref[...]
    a = _leaky(jnp.dot(ha, wg_ref[...],
                       preferred_element_type=jnp.float32) + bg_ref[...])
    b = _leaky(jnp.dot(h * ha, wi_ref[...],
                       preferred_element_type=jnp.float32) + bi_ref[...])
    u = a + b
    sq = jnp.sum(u * u, axis=-1, keepdims=True)
    hn = u * lax.rsqrt(jnp.maximum(sq, 1e-12))
    hn_ref[...] = hn
    hns_ref[...] = hn * dis


def _dense_call(p0, p1, h, dis, wg, bg, wi, bi):
    full = lambda i: (0, 0)
    blk = lambda i: (i, 0)
    return pl.pallas_call(
        _dense_body,
        grid=(GRID,),
        in_specs=[
            pl.BlockSpec((BM, D), blk),
            pl.BlockSpec((BM, D), blk),
            pl.BlockSpec((BM, D), blk),
            pl.BlockSpec((BM, D), blk),
            pl.BlockSpec((D, D), full),
            pl.BlockSpec((1, D), full),
            pl.BlockSpec((D, D), full),
            pl.BlockSpec((1, D), full),
        ],
        out_specs=[
            pl.BlockSpec((BM, D), blk),
            pl.BlockSpec((BM, D), blk),
        ],
        out_shape=[
            jax.ShapeDtypeStruct((N, D), jnp.float32),
            jax.ShapeDtypeStruct((N, D), jnp.float32),
        ],
    )(p0, p1, h, dis, wg, bg, wi, bi)


@jax.jit
def _run(x, edge_index, Wg0, bg0, Wi0, bi0, Wg1, bg1, Wi1, bi1,
         Wg2, bg2, Wi2, bi2):
    row = edge_index[0]
    col = edge_index[1]
    pad = E_PAD - E
    ar = jnp.arange(pad, dtype=jnp.int32)
    row_p = jnp.concatenate([row, N + NZ + ar % NTRASH])
    col_p = jnp.concatenate([col, N + ar % NZ])
    row3 = row_p.reshape(TOT_CH, EB)
    col3 = col_p.reshape(TOT_CH, EB)

    zerosD = jnp.zeros((RPT, D), jnp.float32)

    deg_parts = _make_deg_kernel()(row3)
    dis, hs = _prep_call(deg_parts, x)

    params = [(Wg0, bg0, Wi0, bi0), (Wg1, bg1, Wi1, bi1), (Wg2, bg2, Wi2, bi2)]
    h = x
    outs = [x]
    for (Wg, bg, Wi, bi) in params:
        hs_ext = jnp.concatenate([hs, jnp.zeros((NZ, D), jnp.float32)])
        parts = _make_agg_kernel()(hs_ext, row3, col3, zerosD)
        h, hs = _dense_call(parts[0, :N], parts[1, :N], h, dis,
                            Wg, bg.reshape(1, D), Wi, bi.reshape(1, D))
        outs.append(h)
    return jnp.concatenate(outs, axis=-1)


def kernel(x, edge_index, Wg0, bg0, Wi0, bi0, Wg1, bg1, Wi1, bi1,
           Wg2, bg2, Wi2, bi2):
    return _run(x, edge_index, Wg0, bg0, Wi0, bi0, Wg1, bg1, Wi1, bi1,
                Wg2, bg2, Wi2, bi2)


# speedup vs baseline: 3.0333x; 1.0321x over previous
import functools

import jax
import jax.numpy as jnp
from jax import lax
from jax.experimental import pallas as pl
from jax.experimental.pallas import tpu as pltpu
import jax.experimental.pallas.tpu_sc as plsc

N = 10000
E = 320000
D = 128
NC = 2
NS = 16
NW = NC * NS
EB = 128
NBUF = 2
IB = 4

ITC0 = 80
ITC1 = 80
TOT_CH = NS * (ITC0 + ITC1)
E_PAD = TOT_CH * EB
ITERS = TOT_CH // NW
N_PAD = 10112
RPT = N_PAD // NS
NZ = 8
NTRASH = N_PAD - N - NZ

BM = 2000
GRID = N // BM


def _sc_mesh():
    return plsc.VectorSubcoreMesh(core_axis_name="c", subcore_axis_name="s",
                                  num_cores=NC, num_subcores=NS)


@functools.lru_cache(maxsize=None)
def _make_deg_kernel():
    @functools.partial(
        pl.kernel,
        out_type=jax.ShapeDtypeStruct((NW, N_PAD), jnp.float32),
        mesh=_sc_mesh(),
        compiler_params=pltpu.CompilerParams(needs_layout_passes=False),
        scratch_types=[
            pltpu.VMEM((ITERS, EB), jnp.int32),
            pltpu.VMEM((N_PAD,), jnp.float32),
        ],
    )
    def _deg_kernel(row_hbm, out_hbm, rowm, hist):
        c = lax.axis_index("c")
        s = lax.axis_index("s")
        wid = s * NC + c
        pltpu.sync_copy(row_hbm.at[pl.ds(wid * ITERS, ITERS)], rowm)

        def zbody(j, carry):
            hist[pl.ds(16 * j, 16)] = jnp.zeros((16,), jnp.float32)
            return carry

        lax.fori_loop(0, N_PAD // 16, zbody, 0)
        ones = jnp.ones((16,), jnp.float32)

        def body(k, carry):
            for j in range(EB // 16):
                i16 = rowm[k, pl.ds(16 * j, 16)]
                plsc.addupdate_scatter(hist, [i16], ones)
            return carry

        lax.fori_loop(0, ITERS, body, 0)
        pltpu.sync_copy(hist, out_hbm.at[wid])

    return _deg_kernel


@functools.lru_cache(maxsize=None)
def _make_agg_kernel():
    @functools.partial(
        pl.kernel,
        out_type=jax.ShapeDtypeStruct((NC, N_PAD, D), jnp.float32),
        mesh=_sc_mesh(),
        scratch_types=(
            [pltpu.VMEM((EB,), jnp.int32) for _ in range(2 * IB)]
            + [pltpu.VMEM((EB, D), jnp.float32) for _ in range(NBUF)]
            + [
                pltpu.VMEM_SHARED((N_PAD, D), jnp.float32),
                pltpu.SemaphoreType.DMA((IB,)),
                pltpu.SemaphoreType.DMA((NBUF,)),
                pltpu.SemaphoreType.DMA((NBUF,)),
            ]
        ),
    )
    def _agg_kernel(hs_hbm, row_hbm, col_hbm, zeros_hbm, out_hbm, *rest):
        rowb = rest[:IB]
        colb = rest[IB:2 * IB]
        gbufs = rest[2 * IB:2 * IB + NBUF]
        acc, isem, gsem, ssem = rest[2 * IB + NBUF:2 * IB + NBUF + 4]
        c = lax.axis_index("c")
        s = lax.axis_index("s")
        base_ch = jnp.where(c == 0, s * ITC0, NS * ITC0 + s * ITC1)
        tg = jnp.where(c == 0, ITC0 // 4, ITC1 // 4)

        def idx_start(k, j):
            pltpu.async_copy(row_hbm.at[base_ch + k], rowb[j], isem.at[j])
            pltpu.async_copy(col_hbm.at[base_ch + k], colb[j], isem.at[j])

        def idx_wait(k, j):
            pltpu.make_async_copy(row_hbm.at[base_ch + k], rowb[j],
                                  isem.at[j]).wait()
            pltpu.make_async_copy(col_hbm.at[base_ch + k], colb[j],
                                  isem.at[j]).wait()

        def gather_start(b, j):
            pltpu.async_copy(hs_hbm.at[colb[j]], gbufs[b], gsem.at[b])

        def gather_wait(b, j):
            pltpu.make_async_copy(hs_hbm.at[colb[j]], gbufs[b],
                                  gsem.at[b]).wait()

        def scat_start(b, j):
            pltpu.async_copy(gbufs[b], acc.at[rowb[j]], ssem.at[b], add=True)

        def scat_wait(b, j):
            pltpu.make_async_copy(gbufs[b], acc.at[rowb[j]],
                                  ssem.at[b]).wait()

        pltpu.sync_copy(zeros_hbm, acc.at[pl.ds(s * RPT, RPT)])
        plsc.subcore_barrier()

        for j in range(IB):
            idx_start(j, j)
        for b in range(NBUF):
            idx_wait(b, b)
            gather_start(b, b)

        def body(t, carry):
            k0 = 4 * t

            def half(base, j0, j1, jn0, jn1, refill_ok, prefetch_ok):
                gather_wait(0, j0)
                scat_start(0, j0)
                gather_wait(1, j1)
                scat_start(1, j1)

                def refill():
                    scat_wait(0, j0)
                    idx_wait(base + 2, jn0)
                    gather_start(0, jn0)
                    scat_wait(1, j1)
                    idx_wait(base + 3, jn1)
                    gather_start(1, jn1)

                if refill_ok is None:
                    refill()
                else:
                    pl.when(refill_ok)(refill)

                @pl.when(prefetch_ok)
                def _():
                    idx_start(base + 4, j0)
                    idx_start(base + 5, j1)

            not_last = t < tg - 1
            half(k0, 0, 1, 2, 3, None, not_last)
            half(k0 + 2, 2, 3, 0, 1, not_last, not_last)
            return carry

        lax.fori_loop(0, tg, body, 0)
        scat_wait(0, 2)
        scat_wait(1, 3)
        plsc.subcore_barrier()
        pltpu.sync_copy(acc.at[pl.ds(s * RPT, RPT)],
                        out_hbm.at[c, pl.ds(s * RPT, RPT)])

    return _agg_kernel


def _prep_body(d_ref, x_ref, dis_ref, h0s_ref):
    deg = lax.dot_general(d_ref[...], jnp.ones((NW, 1), jnp.float32),
                          (((0,), (0,)), ((), ())),
                          preferred_element_type=jnp.float32)[:N]
    dis = jnp.where(deg > 0, lax.rsqrt(jnp.maximum(deg, 1e-12)), 0.0)
    dis_b = jnp.broadcast_to(dis, (N, D))
    dis_ref[...] = dis_b
    h0s_ref[...] = x_ref[...] * dis_b


def _prep_call(d, x):
    return pl.pallas_call(
        _prep_body,
        out_shape=[
            jax.ShapeDtypeStruct((N, D), jnp.float32),
            jax.ShapeDtypeStruct((N, D), jnp.float32),
        ],
    )(d, x)


def _leaky(v):
    return jnp.maximum(v, 0.2 * v)


def _dense_body(p0_ref, p1_ref, h_ref, dis_ref, wg_ref,
                bg_ref, wi_ref, bi_ref, hn_ref, hns_ref):
    dis = dis_ref[...]
    ha = (p0_ref[0] + p1_ref[0]) * dis
    h = h_ref[...]
    a = _leaky(jnp.dot(ha, wg_ref[...],
                       preferred_element_type=jnp.float32) + bg_ref[...])
    b = _leaky(jnp.dot(h * ha, wi_ref[...],
                       preferred_element_type=jnp.float32) + bi_ref[...])
    u = a + b
    sq = jnp.sum(u * u, axis=-1, keepdims=True)
    hn = u * lax.rsqrt(jnp.maximum(sq, 1e-12))
    hn_ref[...] = hn
    hns_ref[...] = hn * dis


def _dense_call(p0, p1, h, dis, wg, bg, wi, bi):
    full = lambda i: (0, 0)
    blk = lambda i: (i, 0)
    return pl.pallas_call(
        _dense_body,
        grid=(GRID,),
        in_specs=[
            pl.BlockSpec((1, BM, D), lambda i: (0, i, 0)),
            pl.BlockSpec((1, BM, D), lambda i: (1, i, 0)),
            pl.BlockSpec((BM, D), blk),
            pl.BlockSpec((BM, D), blk),
            pl.BlockSpec((D, D), full),
            pl.BlockSpec((1, D), full),
            pl.BlockSpec((D, D), full),
            pl.BlockSpec((1, D), full),
        ],
        out_specs=[
            pl.BlockSpec((BM, D), blk),
            pl.BlockSpec((BM, D), blk),
        ],
        out_shape=[
            jax.ShapeDtypeStruct((N, D), jnp.float32),
            jax.ShapeDtypeStruct((N, D), jnp.float32),
        ],
    )(p0, p1, h, dis, wg, bg, wi, bi)


@jax.jit
def _run(x, edge_index, Wg0, bg0, Wi0, bi0, Wg1, bg1, Wi1, bi1,
         Wg2, bg2, Wi2, bi2):
    row = edge_index[0]
    col = edge_index[1]
    pad = E_PAD - E
    ar = jnp.arange(pad, dtype=jnp.int32)
    row_p = jnp.concatenate([row, N + NZ + ar % NTRASH])
    col_p = jnp.concatenate([col, N + ar % NZ])
    row3 = row_p.reshape(TOT_CH, EB)
    col3 = col_p.reshape(TOT_CH, EB)

    zerosD = jnp.zeros((RPT, D), jnp.float32)

    deg_parts = _make_deg_kernel()(row3)
    dis, hs = _prep_call(deg_parts, x)

    params = [(Wg0, bg0, Wi0, bi0), (Wg1, bg1, Wi1, bi1), (Wg2, bg2, Wi2, bi2)]
    h = x
    outs = [x]
    for (Wg, bg, Wi, bi) in params:
        hs_ext = jnp.concatenate([hs, jnp.zeros((NZ, D), jnp.float32)])
        parts = _make_agg_kernel()(hs_ext, row3, col3, zerosD)
        h, hs = _dense_call(parts, parts, h, dis,
                            Wg, bg.reshape(1, D), Wi, bi.reshape(1, D))
        outs.append(h)
    return jnp.concatenate(outs, axis=-1)


def kernel(x, edge_index, Wg0, bg0, Wi0, bi0, Wg1, bg1, Wi1, bi1,
           Wg2, bg2, Wi2, bi2):
    return _run(x, edge_index, Wg0, bg0, Wi0, bi0, Wg1, bg1, Wi1, bi1,
                Wg2, bg2, Wi2, bi2)


# speedup vs baseline: 3.2085x; 1.0578x over previous
import functools

import jax
import jax.numpy as jnp
from jax import lax
from jax.experimental import pallas as pl
from jax.experimental.pallas import tpu as pltpu
import jax.experimental.pallas.tpu_sc as plsc

N = 10000
E = 320000
D = 128
NC = 2
NS = 16
NW = NC * NS
EB = 128
NBUF = 2
IB = 4

ITC0 = 80
ITC1 = 80
TOT_CH = NS * (ITC0 + ITC1)
E_PAD = TOT_CH * EB
ITERS = TOT_CH // NW
N_PAD = 10112
RPT = N_PAD // NS
NZ = 8
NTRASH = N_PAD - N - NZ

BM = 2000
GRID = N // BM


def _sc_mesh():
    return plsc.VectorSubcoreMesh(core_axis_name="c", subcore_axis_name="s",
                                  num_cores=NC, num_subcores=NS)


@functools.lru_cache(maxsize=None)
def _make_deg_kernel():
    @functools.partial(
        pl.kernel,
        out_type=jax.ShapeDtypeStruct((NW, N_PAD), jnp.float32),
        mesh=_sc_mesh(),
        compiler_params=pltpu.CompilerParams(needs_layout_passes=False),
        scratch_types=[
            pltpu.VMEM((ITERS, EB), jnp.int32),
            pltpu.VMEM((N_PAD,), jnp.float32),
        ],
    )
    def _deg_kernel(row_hbm, out_hbm, rowm, hist):
        c = lax.axis_index("c")
        s = lax.axis_index("s")
        wid = s * NC + c
        pltpu.sync_copy(row_hbm.at[pl.ds(wid * ITERS, ITERS)], rowm)

        def zbody(j, carry):
            hist[pl.ds(16 * j, 16)] = jnp.zeros((16,), jnp.float32)
            return carry

        lax.fori_loop(0, N_PAD // 16, zbody, 0)
        ones = jnp.ones((16,), jnp.float32)

        def body(k, carry):
            for j in range(EB // 16):
                i16 = rowm[k, pl.ds(16 * j, 16)]
                plsc.addupdate_scatter(hist, [i16], ones)
            return carry

        lax.fori_loop(0, ITERS, body, 0)
        pltpu.sync_copy(hist, out_hbm.at[wid])

    return _deg_kernel


@functools.lru_cache(maxsize=None)
def _make_agg_kernel():
    @functools.partial(
        pl.kernel,
        out_type=jax.ShapeDtypeStruct((NC, N_PAD, D), jnp.float32),
        mesh=_sc_mesh(),
        scratch_types=(
            [pltpu.VMEM((EB,), jnp.int32) for _ in range(2 * IB)]
            + [pltpu.VMEM((EB, D), jnp.float32) for _ in range(NBUF)]
            + [
                pltpu.VMEM_SHARED((N_PAD, D), jnp.float32),
                pltpu.SemaphoreType.DMA((IB,)),
                pltpu.SemaphoreType.DMA((NBUF,)),
                pltpu.SemaphoreType.DMA((NBUF,)),
            ]
        ),
    )
    def _agg_kernel(hs_hbm, row_hbm, col_hbm, zeros_hbm, out_hbm, *rest):
        rowb = rest[:IB]
        colb = rest[IB:2 * IB]
        gbufs = rest[2 * IB:2 * IB + NBUF]
        acc, isem, gsem, ssem = rest[2 * IB + NBUF:2 * IB + NBUF + 4]
        c = lax.axis_index("c")
        s = lax.axis_index("s")
        base_ch = jnp.where(c == 0, s * ITC0, NS * ITC0 + s * ITC1)
        tg = jnp.where(c == 0, ITC0 // 4, ITC1 // 4)

        def idx_start(k, j):
            pltpu.async_copy(row_hbm.at[base_ch + k], rowb[j], isem.at[j])
            pltpu.async_copy(col_hbm.at[base_ch + k], colb[j], isem.at[j])

        def idx_wait(k, j):
            pltpu.make_async_copy(row_hbm.at[base_ch + k], rowb[j],
                                  isem.at[j]).wait()
            pltpu.make_async_copy(col_hbm.at[base_ch + k], colb[j],
                                  isem.at[j]).wait()

        def gather_start(b, j):
            pltpu.async_copy(hs_hbm.at[colb[j]], gbufs[b], gsem.at[b])

        def gather_wait(b, j):
            pltpu.make_async_copy(hs_hbm.at[colb[j]], gbufs[b],
                                  gsem.at[b]).wait()

        def scat_start(b, j):
            pltpu.async_copy(gbufs[b], acc.at[rowb[j]], ssem.at[b], add=True)

        def scat_wait(b, j):
            pltpu.make_async_copy(gbufs[b], acc.at[rowb[j]],
                                  ssem.at[b]).wait()

        pltpu.sync_copy(zeros_hbm, acc.at[pl.ds(s * RPT, RPT)])
        plsc.subcore_barrier()

        for j in range(IB):
            idx_start(j, j)
        for b in range(NBUF):
            idx_wait(b, b)
            gather_start(b, b)

        def body(t, carry):
            k0 = 4 * t

            def half(base, j0, j1, jn0, jn1, refill_ok, prefetch_ok):
                gather_wait(0, j0)
                scat_start(0, j0)
                gather_wait(1, j1)
                scat_start(1, j1)

                def refill():
                    scat_wait(0, j0)
                    idx_wait(base + 2, jn0)
                    gather_start(0, jn0)
                    scat_wait(1, j1)
                    idx_wait(base + 3, jn1)
                    gather_start(1, jn1)

                if refill_ok is None:
                    refill()
                else:
                    pl.when(refill_ok)(refill)

                @pl.when(prefetch_ok)
                def _():
                    idx_start(base + 4, j0)
                    idx_start(base + 5, j1)

            not_last = t < tg - 1
            half(k0, 0, 1, 2, 3, None, not_last)
            half(k0 + 2, 2, 3, 0, 1, not_last, not_last)
            return carry

        lax.fori_loop(0, tg, body, 0)
        scat_wait(0, 2)
        scat_wait(1, 3)
        plsc.subcore_barrier()
        pltpu.sync_copy(acc.at[pl.ds(s * RPT, RPT)],
                        out_hbm.at[c, pl.ds(s * RPT, RPT)])

    return _agg_kernel


def _prep_body(d_ref, x_ref, dis_ref, h0s_ref):
    deg = lax.dot_general(d_ref[...], jnp.ones((NW, 1), jnp.float32),
                          (((0,), (0,)), ((), ())),
                          preferred_element_type=jnp.float32)[:N]
    dis = jnp.where(deg > 0, lax.rsqrt(jnp.maximum(deg, 1e-12)), 0.0)
    dis_b = jnp.broadcast_to(dis, (N, D))
    dis_ref[...] = dis_b
    h0s_ref[pl.ds(0, N)] = x_ref[...] * dis_b
    h0s_ref[pl.ds(N, N_PAD - N)] = jnp.zeros((N_PAD - N, D), jnp.float32)


def _prep_call(d, x):
    return pl.pallas_call(
        _prep_body,
        out_shape=[
            jax.ShapeDtypeStruct((N, D), jnp.float32),
            jax.ShapeDtypeStruct((N_PAD, D), jnp.float32),
        ],
    )(d, x)


def _leaky(v):
    return jnp.maximum(v, 0.2 * v)


def _dense_body(p0_ref, p1_ref, h_ref, dis_ref, wg_ref,
                bg_ref, wi_ref, bi_ref, hsprev_ref, hn_ref, hns_ref):
    dis = dis_ref[...]
    ha = (p0_ref[0] + p1_ref[0]) * dis
    h = h_ref[...]
    a = _leaky(jnp.dot(ha, wg_ref[...],
                       preferred_element_type=jnp.float32) + bg_ref[...])
    b = _leaky(jnp.dot(h * ha, wi_ref[...],
                       preferred_element_type=jnp.float32) + bi_ref[...])
    u = a + b
    sq = jnp.sum(u * u, axis=-1, keepdims=True)
    hn = u * lax.rsqrt(jnp.maximum(sq, 1e-12))
    hn_ref[...] = hn
    hns_ref[...] = hn * dis


def _dense_call(p0, p1, h, dis, wg, bg, wi, bi, hs_prev):
    full = lambda i: (0, 0)
    blk = lambda i: (i, 0)
    return pl.pallas_call(
        _dense_body,
        grid=(GRID,),
        in_specs=[
            pl.BlockSpec((1, BM, D), lambda i: (0, i, 0)),
            pl.BlockSpec((1, BM, D), lambda i: (1, i, 0)),
            pl.BlockSpec((BM, D), blk),
            pl.BlockSpec((BM, D), blk),
            pl.BlockSpec((D, D), full),
            pl.BlockSpec((1, D), full),
            pl.BlockSpec((D, D), full),
            pl.BlockSpec((1, D), full),
            pl.BlockSpec(memory_space=pl.ANY),
        ],
        out_specs=[
            pl.BlockSpec((BM, D), blk),
            pl.BlockSpec((BM, D), blk),
        ],
        out_shape=[
            jax.ShapeDtypeStruct((N, D), jnp.float32),
            jax.ShapeDtypeStruct((N_PAD, D), jnp.float32),
        ],
        input_output_aliases={8: 1},
    )(p0, p1, h, dis, wg, bg, wi, bi, hs_prev)


@jax.jit
def _run(x, edge_index, Wg0, bg0, Wi0, bi0, Wg1, bg1, Wi1, bi1,
         Wg2, bg2, Wi2, bi2):
    row = edge_index[0]
    col = edge_index[1]
    pad = E_PAD - E
    ar = jnp.arange(pad, dtype=jnp.int32)
    row_p = jnp.concatenate([row, N + NZ + ar % NTRASH])
    col_p = jnp.concatenate([col, N + ar % NZ])
    row3 = row_p.reshape(TOT_CH, EB)
    col3 = col_p.reshape(TOT_CH, EB)

    zerosD = jnp.zeros((RPT, D), jnp.float32)

    deg_parts = _make_deg_kernel()(row3)
    dis, hs = _prep_call(deg_parts, x)

    params = [(Wg0, bg0, Wi0, bi0), (Wg1, bg1, Wi1, bi1), (Wg2, bg2, Wi2, bi2)]
    h = x
    outs = [x]
    for (Wg, bg, Wi, bi) in params:
        parts = _make_agg_kernel()(hs, row3, col3, zerosD)
        h, hs = _dense_call(parts, parts, h, dis,
                            Wg, bg.reshape(1, D), Wi, bi.reshape(1, D), hs)
        outs.append(h)
    return jnp.concatenate(outs, axis=-1)


def kernel(x, edge_index, Wg0, bg0, Wi0, bi0, Wg1, bg1, Wi1, bi1,
           Wg2, bg2, Wi2, bi2):
    return _run(x, edge_index, Wg0, bg0, Wi0, bi0, Wg1, bg1, Wi1, bi1,
                Wg2, bg2, Wi2, bi2)


# speedup vs baseline: 3.7407x; 1.1659x over previous
import functools

import jax
import jax.numpy as jnp
from jax import lax
from jax.experimental import pallas as pl
from jax.experimental.pallas import tpu as pltpu
import jax.experimental.pallas.tpu_sc as plsc

N = 10000
E = 320000
D = 128
NC = 2
NS = 16
NW = NC * NS
EB = 120
NBUF = 3
IB = 6

ITC0 = 84
ITC1 = 84
TOT_CH = NS * (ITC0 + ITC1)
E_PAD = TOT_CH * EB
ITERS = TOT_CH // NW
N_PAD = 10112
RPT = N_PAD // NS
NZ = 8
NTRASH = N_PAD - N - NZ

BM = 2000
GRID = N // BM


def _sc_mesh():
    return plsc.VectorSubcoreMesh(core_axis_name="c", subcore_axis_name="s",
                                  num_cores=NC, num_subcores=NS)


@functools.lru_cache(maxsize=None)
def _make_deg_kernel():
    @functools.partial(
        pl.kernel,
        out_type=jax.ShapeDtypeStruct((NW, N_PAD), jnp.float32),
        mesh=_sc_mesh(),
        compiler_params=pltpu.CompilerParams(needs_layout_passes=False),
        scratch_types=[
            pltpu.VMEM((ITERS * EB,), jnp.int32),
            pltpu.VMEM((N_PAD,), jnp.float32),
        ],
    )
    def _deg_kernel(row_flat_hbm, out_hbm, rowm, hist):
        c = lax.axis_index("c")
        s = lax.axis_index("s")
        wid = s * NC + c
        pltpu.sync_copy(row_flat_hbm.at[pl.ds(wid * ITERS * EB, ITERS * EB)],
                        rowm)

        def zbody(j, carry):
            hist[pl.ds(16 * j, 16)] = jnp.zeros((16,), jnp.float32)
            return carry

        lax.fori_loop(0, N_PAD // 16, zbody, 0)
        ones = jnp.ones((16,), jnp.float32)

        def body(k, carry):
            for j in range(6):
                i16 = rowm[pl.ds(96 * k + 16 * j, 16)]
                plsc.addupdate_scatter(hist, [i16], ones)
            return carry

        lax.fori_loop(0, ITERS * EB // 96, body, 0)
        pltpu.sync_copy(hist, out_hbm.at[wid])

    return _deg_kernel


@functools.lru_cache(maxsize=None)
def _make_agg_kernel():
    @functools.partial(
        pl.kernel,
        out_type=jax.ShapeDtypeStruct((NC, N_PAD, D), jnp.float32),
        mesh=_sc_mesh(),
        scratch_types=(
            [pltpu.VMEM((EB,), jnp.int32) for _ in range(2 * IB)]
            + [pltpu.VMEM((EB, D), jnp.float32) for _ in range(NBUF)]
            + [
                pltpu.VMEM_SHARED((N_PAD, D), jnp.float32),
                pltpu.SemaphoreType.DMA((IB,)),
                pltpu.SemaphoreType.DMA((NBUF,)),
                pltpu.SemaphoreType.DMA((NBUF,)),
            ]
        ),
    )
    def _agg_kernel(hs_hbm, row_hbm, col_hbm, zeros_hbm, out_hbm, *rest):
        rowb = rest[:IB]
        colb = rest[IB:2 * IB]
        gbufs = rest[2 * IB:2 * IB + NBUF]
        acc, isem, gsem, ssem = rest[2 * IB + NBUF:2 * IB + NBUF + 4]
        c = lax.axis_index("c")
        s = lax.axis_index("s")
        base_ch = jnp.where(c == 0, s * ITC0, NS * ITC0 + s * ITC1)
        tg = jnp.where(c == 0, ITC0 // 4, ITC1 // 4)

        def idx_start(k, j):
            pltpu.async_copy(row_hbm.at[base_ch + k], rowb[j], isem.at[j])
            pltpu.async_copy(col_hbm.at[base_ch + k], colb[j], isem.at[j])

        def idx_wait(k, j):
            pltpu.make_async_copy(row_hbm.at[base_ch + k], rowb[j],
                                  isem.at[j]).wait()
            pltpu.make_async_copy(col_hbm.at[base_ch + k], colb[j],
                                  isem.at[j]).wait()

        def gather_start(b, j):
            pltpu.async_copy(hs_hbm.at[colb[j]], gbufs[b], gsem.at[b])

        def gather_wait(b, j):
            pltpu.make_async_copy(hs_hbm.at[colb[j]], gbufs[b],
                                  gsem.at[b]).wait()

        def scat_start(b, j):
            pltpu.async_copy(gbufs[b], acc.at[rowb[j]], ssem.at[b], add=True)

        def scat_wait(b, j):
            pltpu.make_async_copy(gbufs[b], acc.at[rowb[j]],
                                  ssem.at[b]).wait()

        pltpu.sync_copy(zeros_hbm, acc.at[pl.ds(s * RPT, RPT)])
        plsc.subcore_barrier()

        for j in range(IB):
            idx_start(j, j)
        for b in range(NBUF):
            idx_wait(b, b)
            gather_start(b, b)

        TG = ITC0 // 6

        def body(t, carry):
            k0 = 6 * t
            not_last = t < TG - 1
            for i in range(3):
                gather_wait(i, i)
                scat_start(i, i)
            for i in range(3):
                scat_wait(i, i)
                idx_wait(k0 + 3 + i, 3 + i)
                gather_start(i, 3 + i)

                @pl.when(not_last)
                def _():
                    idx_start(k0 + 6 + i, i)
            for i in range(3):
                gather_wait(i, 3 + i)
                scat_start(i, 3 + i)

            @pl.when(not_last)
            def _():
                for i in range(3):
                    scat_wait(i, 3 + i)
                    idx_wait(k0 + 6 + i, i)
                    gather_start(i, i)
                    idx_start(k0 + 9 + i, 3 + i)

            return carry

        lax.fori_loop(0, TG, body, 0)
        for i in range(3):
            scat_wait(i, 3 + i)
        plsc.subcore_barrier()
        pltpu.sync_copy(acc.at[pl.ds(s * RPT, RPT)],
                        out_hbm.at[c, pl.ds(s * RPT, RPT)])

    return _agg_kernel


def _prep_body(d_ref, x_ref, dis_ref, h0s_ref):
    deg = lax.dot_general(d_ref[...], jnp.ones((NW, 1), jnp.float32),
                          (((0,), (0,)), ((), ())),
                          preferred_element_type=jnp.float32)[:N]
    dis = jnp.where(deg > 0, lax.rsqrt(jnp.maximum(deg, 1e-12)), 0.0)
    dis_b = jnp.broadcast_to(dis, (N, D))
    dis_ref[...] = dis_b
    h0s_ref[pl.ds(0, N)] = x_ref[...] * dis_b
    h0s_ref[pl.ds(N, N_PAD - N)] = jnp.zeros((N_PAD - N, D), jnp.float32)


def _prep_call(d, x):
    return pl.pallas_call(
        _prep_body,
        out_shape=[
            jax.ShapeDtypeStruct((N, D), jnp.float32),
            jax.ShapeDtypeStruct((N_PAD, D), jnp.float32),
        ],
    )(d, x)


def _leaky(v):
    return jnp.maximum(v, 0.2 * v)


def _dense_body(p0_ref, p1_ref, h_ref, dis_ref, wg_ref,
                bg_ref, wi_ref, bi_ref, hsprev_ref, hn_ref, hns_ref):
    dis = dis_ref[...]
    ha = (p0_ref[0] + p1_ref[0]) * dis
    h = h_ref[...]
    a = _leaky(jnp.dot(ha, wg_ref[...],
                       preferred_element_type=jnp.float32) + bg_ref[...])
    b = _leaky(jnp.dot(h * ha, wi_ref[...],
                       preferred_element_type=jnp.float32) + bi_ref[...])
    u = a + b
    sq = jnp.sum(u * u, axis=-1, keepdims=True)
    hn = u * lax.rsqrt(jnp.maximum(sq, 1e-12))
    hn_ref[...] = hn
    hns_ref[...] = hn * dis


def _dense_call(p0, p1, h, dis, wg, bg, wi, bi, hs_prev):
    full = lambda i: (0, 0)
    blk = lambda i: (i, 0)
    return pl.pallas_call(
        _dense_body,
        grid=(GRID,),
        in_specs=[
            pl.BlockSpec((1, BM, D), lambda i: (0, i, 0)),
            pl.BlockSpec((1, BM, D), lambda i: (1, i, 0)),
            pl.BlockSpec((BM, D), blk),
            pl.BlockSpec((BM, D), blk),
            pl.BlockSpec((D, D), full),
            pl.BlockSpec((1, D), full),
            pl.BlockSpec((D, D), full),
            pl.BlockSpec((1, D), full),
            pl.BlockSpec(memory_space=pl.ANY),
        ],
        out_specs=[
            pl.BlockSpec((BM, D), blk),
            pl.BlockSpec((BM, D), blk),
        ],
        out_shape=[
            jax.ShapeDtypeStruct((N, D), jnp.float32),
            jax.ShapeDtypeStruct((N_PAD, D), jnp.float32),
        ],
        input_output_aliases={8: 1},
    )(p0, p1, h, dis, wg, bg, wi, bi, hs_prev)


@jax.jit
def _run(x, edge_index, Wg0, bg0, Wi0, bi0, Wg1, bg1, Wi1, bi1,
         Wg2, bg2, Wi2, bi2):
    row = edge_index[0]
    col = edge_index[1]
    pad = E_PAD - E
    ar = jnp.arange(pad, dtype=jnp.int32)
    row_p = jnp.concatenate([row, N + NZ + ar % NTRASH])
    col_p = jnp.concatenate([col, N + ar % NZ])
    row3 = row_p.reshape(TOT_CH, EB)
    col3 = col_p.reshape(TOT_CH, EB)


    zerosD = jnp.zeros((RPT, D), jnp.float32)

    deg_parts = _make_deg_kernel()(row_p)
    dis, hs = _prep_call(deg_parts, x)

    params = [(Wg0, bg0, Wi0, bi0), (Wg1, bg1, Wi1, bi1), (Wg2, bg2, Wi2, bi2)]
    h = x
    outs = [x]
    for (Wg, bg, Wi, bi) in params:
        parts = _make_agg_kernel()(hs, row3, col3, zerosD)
        h, hs = _dense_call(parts, parts, h, dis,
                            Wg, bg.reshape(1, D), Wi, bi.reshape(1, D), hs)
        outs.append(h)
    return jnp.concatenate(outs, axis=-1)


def kernel(x, edge_index, Wg0, bg0, Wi0, bi0, Wg1, bg1, Wi1, bi1,
           Wg2, bg2, Wi2, bi2):
    return _run(x, edge_index, Wg0, bg0, Wi0, bi0, Wg1, bg1, Wi1, bi1,
                Wg2, bg2, Wi2, bi2)
